# initial kernel scaffold (unmeasured)
import os

import jax
import jax.numpy as jnp
from jax import lax
from jax.experimental import pallas as pl
from jax.experimental.pallas import tpu as pltpu

N_DEV = 8
_INTERPRET = os.environ.get("KERNEL_INTERPRET", "0") == "1"


def _mm(a, b):
    return lax.dot_general(
        a, b, (((1,), (0,)), ((), ())), preferred_element_type=jnp.float32
    )


def kernel(x, w_mat, scale_x, scale_w):
    m_glob, k_loc = x.shape
    k_glob, n_out = w_mat.shape
    mb = m_glob // N_DEV

    def body(x_ref, w_ref, sx_ref, sw_ref, out_ref, x8_ref, comm_ref,
             send_sems, recv_sems):
        me = lax.axis_index("i")

        x8_ref[...] = x_ref[...].astype(jnp.float8_e4m3fn)

        barrier = pltpu.get_barrier_semaphore()
        for d in range(1, N_DEV):
            pl.semaphore_signal(
                barrier, inc=1,
                device_id=((me + d) % N_DEV,),
                device_id_type=pl.DeviceIdType.MESH,
            )
        pl.semaphore_wait(barrier, N_DEV - 1)

        rdmas = []
        for d in range(1, N_DEV):
            dst = (me + d) % N_DEV
            rdma = pltpu.make_async_remote_copy(
                src_ref=x8_ref.at[pl.ds(dst * mb, mb), :],
                dst_ref=comm_ref.at[d - 1],
                send_sem=send_sems.at[d - 1],
                recv_sem=recv_sems.at[d - 1],
                device_id=(dst,),
                device_id_type=pl.DeviceIdType.MESH,
            )
            rdma.start()
            rdmas.append(rdma)

        def wblk(kblk):
            return w_ref[pl.ds(kblk * k_loc, k_loc), :].astype(jnp.bfloat16)

        xloc = x8_ref[pl.ds(me * mb, mb), :].astype(jnp.bfloat16)
        out_ref[...] = _mm(xloc, wblk(me))

        for d in range(1, N_DEV):
            src = (me - d) % N_DEV
            rdmas[d - 1].wait_recv()
            out_ref[...] += _mm(comm_ref[d - 1].astype(jnp.bfloat16), wblk(src))

        for d in range(1, N_DEV):
            rdmas[d - 1].wait_send()

        y = out_ref[...] * (sx_ref[0] * sw_ref[0])
        out_ref[...] = y * jax.nn.sigmoid(jnp.clip(y, -60.0, 60.0))

    return pl.pallas_call(
        body,
        out_shape=jax.ShapeDtypeStruct((mb, n_out), jnp.float32),
        in_specs=[
            pl.BlockSpec(memory_space=pltpu.VMEM),
            pl.BlockSpec(memory_space=pltpu.VMEM),
            pl.BlockSpec(memory_space=pltpu.SMEM),
            pl.BlockSpec(memory_space=pltpu.SMEM),
        ],
        out_specs=pl.BlockSpec(memory_space=pltpu.VMEM),
        scratch_shapes=[
            pltpu.VMEM((m_glob, k_loc), jnp.float8_e4m3fn),
            pltpu.VMEM((N_DEV - 1, mb, k_loc), jnp.float8_e4m3fn),
            pltpu.SemaphoreType.DMA((N_DEV - 1,)),
            pltpu.SemaphoreType.DMA((N_DEV - 1,)),
        ],
        compiler_params=pltpu.CompilerParams(collective_id=0),
        interpret=_INTERPRET,
    )(x, w_mat, scale_x, scale_w)


# baseline (device time: 45440 ns/iter reference)
import os

import jax
import jax.numpy as jnp
from jax import lax
from jax.experimental import pallas as pl
from jax.experimental.pallas import tpu as pltpu

N_DEV = 8
_INTERPRET = os.environ.get("KERNEL_INTERPRET", "0") == "1"


def _mm(a, b):
    return lax.dot_general(
        a, b, (((1,), (0,)), ((), ())), preferred_element_type=jnp.float32
    )


def kernel(x, w_mat, scale_x, scale_w):
    m_glob, k_loc = x.shape
    k_glob, n_out = w_mat.shape
    mb = m_glob // N_DEV

    def body(x_ref, w_ref, sx_ref, sw_ref, out_ref, x8_ref, comm_ref,
             send_sems, recv_sems):
        me = lax.axis_index("i")

        x8_ref[...] = x_ref[...].astype(jnp.float8_e4m3fn)

        barrier = pltpu.get_barrier_semaphore()
        for d in range(1, N_DEV):
            pl.semaphore_signal(
                barrier, inc=1,
                device_id=((me + d) % N_DEV,),
                device_id_type=pl.DeviceIdType.MESH,
            )
        pl.semaphore_wait(barrier, N_DEV - 1)

        rdmas = []
        for d in range(1, N_DEV):
            dst = (me + d) % N_DEV
            rdma = pltpu.make_async_remote_copy(
                src_ref=x8_ref.at[pl.ds(dst * mb, mb), :],
                dst_ref=comm_ref.at[d - 1],
                send_sem=send_sems.at[d - 1],
                recv_sem=recv_sems.at[d - 1],
                device_id=(dst,),
                device_id_type=pl.DeviceIdType.MESH,
            )
            rdma.start()
            rdmas.append(rdma)

        def wblk(kblk):
            return w_ref[pl.ds(kblk * k_loc, k_loc), :].astype(jnp.bfloat16)

        xloc = x8_ref[pl.ds(me * mb, mb), :].astype(jnp.bfloat16)
        out_ref[...] = _mm(xloc, wblk(me))

        for d in range(1, N_DEV):
            src = (me - d) % N_DEV
            rdmas[d - 1].wait_recv()
            out_ref[...] += _mm(comm_ref[d - 1].astype(jnp.bfloat16), wblk(src))

        for d in range(1, N_DEV):
            rdmas[d - 1].wait_send()

        y = out_ref[...] * (sx_ref[0] * sw_ref[0])
        out_ref[...] = y * jax.nn.sigmoid(jnp.clip(y, -60.0, 60.0))

    return pl.pallas_call(
        body,
        out_shape=jax.ShapeDtypeStruct((mb, n_out), jnp.float32),
        in_specs=[
            pl.BlockSpec(memory_space=pltpu.VMEM),
            pl.BlockSpec(memory_space=pltpu.VMEM),
            pl.BlockSpec(memory_space=pltpu.SMEM),
            pl.BlockSpec(memory_space=pltpu.SMEM),
        ],
        out_specs=pl.BlockSpec(memory_space=pltpu.VMEM),
        scratch_shapes=[
            pltpu.VMEM((m_glob, k_loc), jnp.float8_e4m3fn),
            pltpu.VMEM((N_DEV - 1, mb, k_loc), jnp.float8_e4m3fn),
            pltpu.SemaphoreType.DMA((N_DEV - 1,)),
            pltpu.SemaphoreType.DMA((N_DEV - 1,)),
        ],
        compiler_params=pltpu.CompilerParams(
            collective_id=0, vmem_limit_bytes=100 * 1024 * 1024
        ),
        interpret=pltpu.InterpretParams() if _INTERPRET else False,
    )(x, w_mat, scale_x, scale_w)


# device time: 34453 ns/iter; 1.3189x vs baseline; 1.3189x over previous
import os

import jax
import jax.numpy as jnp
from jax import lax
from jax.experimental import pallas as pl
from jax.experimental.pallas import tpu as pltpu

N_DEV = 8
_INTERPRET = os.environ.get("KERNEL_INTERPRET", "0") == "1"
_MM_DTYPE = jnp.bfloat16 if os.environ.get("KERNEL_MM_BF16") == "1" else jnp.float8_e4m3fn


def _mm(a, b):
    return lax.dot_general(
        a, b, (((1,), (0,)), ((), ())), preferred_element_type=jnp.float32
    )


def kernel(x, w_mat, scale_x, scale_w):
    m_glob, k_loc = x.shape
    k_glob, n_out = w_mat.shape
    mb = m_glob // N_DEV

    def body(x_ref, w_hbm, sx_ref, sw_ref, out_ref,
             x8_ref, comm_ref, wf_ref, send_sems, recv_sems, wdma_sems):
        me = lax.axis_index("i")

        def kblk(d):
            return (me - d) % N_DEV

        def start_wdma(step, slot):
            cp = pltpu.make_async_copy(
                w_hbm.at[pl.ds(kblk(step) * k_loc, k_loc), :],
                wf_ref.at[slot],
                wdma_sems.at[slot],
            )
            cp.start()
            return cp

        def wcast(slot):
            return wf_ref[slot].astype(_MM_DTYPE)

        wcp = [start_wdma(0, 0)]
        x8_ref[...] = x_ref[...].astype(jnp.float8_e4m3fn)

        barrier = pltpu.get_barrier_semaphore()
        for d in range(1, N_DEV):
            pl.semaphore_signal(
                barrier, inc=1,
                device_id=((me + d) % N_DEV,),
                device_id_type=pl.DeviceIdType.MESH,
            )
        pl.semaphore_wait(barrier, N_DEV - 1)

        rdmas = []
        for d in range(1, N_DEV):
            dst = (me + d) % N_DEV
            rdma = pltpu.make_async_remote_copy(
                src_ref=x8_ref.at[pl.ds(dst * mb, mb), :],
                dst_ref=comm_ref.at[d - 1],
                send_sem=send_sems.at[d - 1],
                recv_sem=recv_sems.at[d - 1],
                device_id=(dst,),
                device_id_type=pl.DeviceIdType.MESH,
            )
            rdma.start()
            rdmas.append(rdma)

        wcp.append(start_wdma(1, 1))

        wcp[0].wait()
        out_ref[...] = _mm(
            x8_ref[pl.ds(me * mb, mb), :].astype(_MM_DTYPE), wcast(0)
        )

        for d in range(1, N_DEV):
            slot = d % 2
            if d + 1 < N_DEV:
                wcp.append(start_wdma(d + 1, (d + 1) % 2))
            rdmas[d - 1].wait_recv()
            wcp[d].wait()
            out_ref[...] += _mm(comm_ref[d - 1].astype(_MM_DTYPE), wcast(slot))

        for d in range(1, N_DEV):
            rdmas[d - 1].wait_send()

        y = out_ref[...] * (sx_ref[0] * sw_ref[0])
        out_ref[...] = y * jax.nn.sigmoid(jnp.clip(y, -60.0, 60.0))

    return pl.pallas_call(
        body,
        out_shape=jax.ShapeDtypeStruct((mb, n_out), jnp.float32),
        in_specs=[
            pl.BlockSpec(memory_space=pltpu.VMEM),
            pl.BlockSpec(memory_space=pltpu.MemorySpace.HBM),
            pl.BlockSpec(memory_space=pltpu.SMEM),
            pl.BlockSpec(memory_space=pltpu.SMEM),
        ],
        out_specs=pl.BlockSpec(memory_space=pltpu.VMEM),
        scratch_shapes=[
            pltpu.VMEM((m_glob, k_loc), jnp.float8_e4m3fn),
            pltpu.VMEM((N_DEV - 1, mb, k_loc), jnp.float8_e4m3fn),
            pltpu.VMEM((2, k_loc, n_out), jnp.float32),
            pltpu.SemaphoreType.DMA((N_DEV - 1,)),
            pltpu.SemaphoreType.DMA((N_DEV - 1,)),
            pltpu.SemaphoreType.DMA((2,)),
        ],
        compiler_params=pltpu.CompilerParams(
            collective_id=0, vmem_limit_bytes=100 * 1024 * 1024
        ),
        interpret=pltpu.InterpretParams() if _INTERPRET else False,
    )(x, w_mat, scale_x, scale_w)
